# baseline (speedup 1.0000x reference)
import functools

import jax
import jax.numpy as jnp
from jax import lax
from jax.experimental import pallas as pl
from jax.experimental.pallas import tpu as pltpu
from jax.experimental.pallas import tpu_sc as plsc

_NBINS = 256
_B, _H, _W, _C = 64, 512, 512, 3
_PIX = _H * _W
_NC, _NS, _L = 2, 16, 16
_NW = _NC * _NS
_IMGS_PER_W = _B // _NW
_CHUNK_PX = 16384
_CHUNK_F = _CHUNK_PX * _C
_CHUNKS_PER_IMG = _PIX // _CHUNK_PX
_TOTAL_CHUNKS = _CHUNKS_PER_IMG * _IMGS_PER_W
_GROUPS = _CHUNK_PX // _L
_HIST_W = _IMGS_PER_W * _L * _NBINS
_INV = 1.0 / float(_PIX)

_W0 = 0.2989 * _NBINS
_W1 = 0.5870 * _NBINS
_W2 = 0.1140 * _NBINS

_mesh = plsc.VectorSubcoreMesh(
    core_axis_name="c", subcore_axis_name="s",
    num_cores=_NC, num_subcores=_NS)


@functools.partial(
    pl.kernel,
    out_type=jax.ShapeDtypeStruct((_B * _NBINS,), jnp.float32),
    mesh=_mesh,
    scratch_types=[
        pltpu.VMEM((_CHUNK_F,), jnp.float32),
        pltpu.VMEM((_CHUNK_F,), jnp.float32),
        pltpu.VMEM((_HIST_W,), jnp.float32),
        pltpu.VMEM((_NBINS,), jnp.float32),
        pltpu.SemaphoreType.DMA,
        pltpu.SemaphoreType.DMA,
    ],
    compiler_params=pltpu.CompilerParams(needs_layout_passes=False),
)
def _hist_sc(x_hbm, out_hbm, buf0, buf1, hist, ostage, sem0, sem1):
    cid = lax.axis_index("c")
    sid = lax.axis_index("s")
    wid = sid * _NC + cid
    img0 = wid * _IMGS_PER_W
    fbase = img0 * (_PIX * _C)

    zeros = jnp.zeros((_L,), jnp.float32)
    ones = jnp.ones((_L,), jnp.float32)
    iota = lax.iota(jnp.int32, _L)
    iota3 = iota * 3
    lane_base = iota * _NBINS

    def _zbody(i, carry):
        hist[pl.ds(pl.multiple_of(i * _L, _L), _L)] = zeros
        return carry
    lax.fori_loop(0, _HIST_W // _L, _zbody, 0)

    bufs = (buf0, buf1)
    sems = (sem0, sem1)

    def _start(c, b):
        off = pl.multiple_of(fbase + c * _CHUNK_F, _CHUNK_F)
        pltpu.make_async_copy(
            x_hbm.at[pl.ds(off, _CHUNK_F)], bufs[b], sems[b]).start()

    def _process(c, b):
        buf = bufs[b]
        pltpu.make_async_copy(
            x_hbm.at[pl.ds(0, _CHUNK_F)], buf, sems[b]).wait()
        hbase = lane_base + jnp.where(
            c >= _CHUNKS_PER_IMG, _L * _NBINS, 0)

        def _gbody(g, carry):
            ir = iota3 + g * (_L * 3)
            r = plsc.load_gather(buf, [ir])
            gc = plsc.load_gather(buf, [ir + 1])
            bc = plsc.load_gather(buf, [ir + 2])
            acc = r * _W0 + gc * _W1 + bc * _W2
            bin_ = jnp.minimum(
                jnp.maximum(acc.astype(jnp.int32), 0), _NBINS - 1)
            plsc.addupdate_scatter(hist, [hbase + bin_], ones)
            return carry
        lax.fori_loop(0, _GROUPS, _gbody, 0, unroll=4)

    _start(0, 0)

    def _cbody(c2, carry):
        c = c2 * 2
        _start(c + 1, 1)
        _process(c, 0)

        @pl.when(c2 < _TOTAL_CHUNKS // 2 - 1)
        def _():
            _start(c + 2, 0)
        _process(c + 1, 1)
        return carry
    lax.fori_loop(0, _TOTAL_CHUNKS // 2, _cbody, 0)

    for img_local in range(_IMGS_PER_W):
        def _rbody(g, carry, img_local=img_local):
            gb = pl.multiple_of(g * _L, _L)
            acc = zeros
            for l in range(_L):
                off = pl.multiple_of(
                    img_local * (_L * _NBINS) + l * _NBINS + gb, _L)
                acc = acc + hist[pl.ds(off, _L)]
            ostage[pl.ds(gb, _L)] = acc * _INV
            return carry
        lax.fori_loop(0, _NBINS // _L, _rbody, 0)
        oimg = img0 + img_local
        pltpu.sync_copy(
            ostage, out_hbm.at[pl.ds(pl.multiple_of(oimg * _NBINS, _NBINS),
                                     _NBINS)])


def kernel(inputs):
    x = inputs.astype(jnp.float32).reshape(-1)
    out = _hist_sc(x)
    return out.reshape(_B, _NBINS)


# speedup vs baseline: 54.0457x; 54.0457x over previous
import functools

import jax
import jax.numpy as jnp
import numpy as np
from jax import lax
from jax.experimental import pallas as pl
from jax.experimental.pallas import tpu as pltpu
from jax.experimental.pallas import tpu_sc as plsc

_NBINS = 256
_B, _H, _W, _C = 64, 512, 512, 3
_PIX = _H * _W
_NC, _NS, _L = 2, 16, 16
_NW = _NC * _NS
_IMGS_PER_W = _B // _NW
_CHUNK_PX = 16384
_CHUNKS_PER_IMG = _PIX // _CHUNK_PX
_TOTAL_CHUNKS = _CHUNKS_PER_IMG * _IMGS_PER_W
_PAIRS = _CHUNK_PX // (2 * _L)
_HIST_W = _IMGS_PER_W * _L * _NBINS
_INV = 1.0 / float(_PIX)


def _bf16_rne(v: float) -> float:
    u = np.float32(v).view(np.uint32)
    u = (u + np.uint32(0x7FFF) + ((u >> np.uint32(16)) & np.uint32(1)))
    return float((u & np.uint32(0xFFFF0000)).view(np.float32))


_W0 = _bf16_rne(0.2989) * _NBINS
_W1 = _bf16_rne(0.5870) * _NBINS
_W2 = _bf16_rne(0.1140) * _NBINS

_mesh = plsc.VectorSubcoreMesh(
    core_axis_name="c", subcore_axis_name="s",
    num_cores=_NC, num_subcores=_NS)


@functools.partial(
    pl.kernel,
    out_type=jax.ShapeDtypeStruct((_B * _NBINS,), jnp.float32),
    mesh=_mesh,
    scratch_types=[
        pltpu.VMEM((_CHUNK_PX,), jnp.float32),
        pltpu.VMEM((_CHUNK_PX,), jnp.float32),
        pltpu.VMEM((_CHUNK_PX,), jnp.float32),
        pltpu.VMEM((_CHUNK_PX,), jnp.float32),
        pltpu.VMEM((_CHUNK_PX,), jnp.float32),
        pltpu.VMEM((_CHUNK_PX,), jnp.float32),
        pltpu.VMEM((_HIST_W,), jnp.float32),
        pltpu.VMEM((_NBINS,), jnp.float32),
        pltpu.SemaphoreType.DMA,
        pltpu.SemaphoreType.DMA,
    ],
    compiler_params=pltpu.CompilerParams(needs_layout_passes=False),
)
def _hist_sc(x_hbm, out_hbm, br0, bg0, bb0, br1, bg1, bb1, hist, ostage,
             sem0, sem1):
    cid = lax.axis_index("c")
    sid = lax.axis_index("s")
    wid = sid * _NC + cid
    img0 = wid * _IMGS_PER_W
    fbase = img0 * (_PIX * _C)

    zeros = jnp.zeros((_L,), jnp.float32)
    ones = jnp.ones((_L,), jnp.float32)
    iota = lax.iota(jnp.int32, _L)
    lane_base = iota * _NBINS

    def _zbody(i, carry):
        hist[pl.ds(pl.multiple_of(i * _L, _L), _L)] = zeros
        return carry
    lax.fori_loop(0, _HIST_W // _L, _zbody, 0)

    sems = (sem0, sem1)
    bufsets = ((br0, bg0, bb0), (br1, bg1, bb1))

    def _plane_off(c, plane):
        img_l = jnp.where(c >= _CHUNKS_PER_IMG, 1, 0)
        ci = c - img_l * _CHUNKS_PER_IMG
        return pl.multiple_of(
            fbase + img_l * (_PIX * _C) + plane * _PIX + ci * _CHUNK_PX,
            _CHUNK_PX)

    def _start(c, b):
        for plane, buf in enumerate(bufsets[b]):
            pltpu.make_async_copy(
                x_hbm.at[pl.ds(_plane_off(c, plane), _CHUNK_PX)],
                buf, sems[b]).start()

    def _wait(b):
        for buf in bufsets[b]:
            pltpu.make_async_copy(
                x_hbm.at[pl.ds(0, _CHUNK_PX)], buf, sems[b]).wait()

    def _process(c, b):
        _wait(b)
        br, bg, bb = bufsets[b]
        hbase = lane_base + jnp.where(
            c >= _CHUNKS_PER_IMG, _L * _NBINS, 0)

        def _rne(v):
            u = plsc.bitcast(v, jnp.uint32)
            u = u + jnp.uint32(0x7FFF) + ((u >> jnp.uint32(16))
                                          & jnp.uint32(1))
            return plsc.bitcast(u & jnp.uint32(0xFFFF0000), jnp.float32)

        def _gbody(g, carry):
            o = pl.multiple_of(g * (2 * _L), 2 * _L)
            o1 = pl.multiple_of(o + _L, _L)
            r0, r1 = _rne(br[pl.ds(o, _L)]), _rne(br[pl.ds(o1, _L)])
            g0, g1 = _rne(bg[pl.ds(o, _L)]), _rne(bg[pl.ds(o1, _L)])
            b0, b1 = _rne(bb[pl.ds(o, _L)]), _rne(bb[pl.ds(o1, _L)])
            acc0 = (r0 * _W0 + g0 * _W1) + b0 * _W2
            acc1 = (r1 * _W0 + g1 * _W1) + b1 * _W2
            i0 = jnp.minimum(
                jnp.maximum(acc0.astype(jnp.int32), 0), _NBINS - 1)
            i1 = jnp.minimum(
                jnp.maximum(acc1.astype(jnp.int32), 0), _NBINS - 1)
            plsc.addupdate_scatter(hist, [hbase + i0], ones)
            plsc.addupdate_scatter(hist, [hbase + i1], ones)
            return carry
        lax.fori_loop(0, _PAIRS, _gbody, 0, unroll=4)

    _start(0, 0)

    def _cbody(c2, carry):
        c = c2 * 2
        _start(c + 1, 1)
        _process(c, 0)

        @pl.when(c2 < _TOTAL_CHUNKS // 2 - 1)
        def _():
            _start(c + 2, 0)
        _process(c + 1, 1)
        return carry
    lax.fori_loop(0, _TOTAL_CHUNKS // 2, _cbody, 0)

    for img_local in range(_IMGS_PER_W):
        def _rbody(g, carry, img_local=img_local):
            gb = pl.multiple_of(g * _L, _L)
            acc = zeros
            for l in range(_L):
                off = pl.multiple_of(
                    img_local * (_L * _NBINS) + l * _NBINS + gb, _L)
                acc = acc + hist[pl.ds(off, _L)]
            ostage[pl.ds(gb, _L)] = acc * _INV
            return carry
        lax.fori_loop(0, _NBINS // _L, _rbody, 0)
        oimg = img0 + img_local
        pltpu.sync_copy(
            ostage, out_hbm.at[pl.ds(pl.multiple_of(oimg * _NBINS, _NBINS),
                                     _NBINS)])


def kernel(inputs):
    x = inputs.astype(jnp.float32)
    y = x.transpose(0, 3, 1, 2)
    y = y.reshape(_B, _C, _H // 8, 8, _W // 128, 128)
    y = y.transpose(0, 1, 2, 4, 3, 5)
    flat = y.reshape(-1)
    out = _hist_sc(flat)
    return out.reshape(_B, _NBINS)


# speedup vs baseline: 172.5886x; 3.1934x over previous
import functools

import jax
import jax.numpy as jnp
import numpy as np
from jax import lax
from jax.experimental import pallas as pl
from jax.experimental.pallas import tpu as pltpu
from jax.experimental.pallas import tpu_sc as plsc

_NBINS = 256
_B, _H, _W, _C = 64, 512, 512, 3
_PIX = _H * _W
_NC, _NS, _L = 2, 16, 16
_NW = _NC * _NS
_IMGS_PER_W = _B // _NW
_CHUNK_PX = 16384
_CHUNKS_PER_IMG = _PIX // _CHUNK_PX
_TOTAL_CHUNKS = _CHUNKS_PER_IMG * _IMGS_PER_W
_PAIRS = _CHUNK_PX // (2 * _L)
_HIST_W = _IMGS_PER_W * _L * _NBINS
_INV = 1.0 / float(_PIX)


def _bf16_rne(v: float) -> float:
    u = np.float32(v).view(np.uint32)
    u = (u + np.uint32(0x7FFF) + ((u >> np.uint32(16)) & np.uint32(1)))
    return float((u & np.uint32(0xFFFF0000)).view(np.float32))


_W0 = _bf16_rne(0.2989) * _NBINS
_W1 = _bf16_rne(0.5870) * _NBINS
_W2 = _bf16_rne(0.1140) * _NBINS

_MAGIC = float(2.0 ** 23) - 0.5

_mesh = plsc.VectorSubcoreMesh(
    core_axis_name="c", subcore_axis_name="s",
    num_cores=_NC, num_subcores=_NS)


@functools.partial(
    pl.kernel,
    out_type=jax.ShapeDtypeStruct((_B * _NBINS,), jnp.float32),
    mesh=_mesh,
    scratch_types=[
        pltpu.VMEM((_CHUNK_PX,), jnp.float32),
        pltpu.VMEM((_CHUNK_PX,), jnp.float32),
        pltpu.VMEM((_CHUNK_PX,), jnp.float32),
        pltpu.VMEM((_CHUNK_PX,), jnp.float32),
        pltpu.VMEM((_CHUNK_PX,), jnp.float32),
        pltpu.VMEM((_CHUNK_PX,), jnp.float32),
        pltpu.VMEM((_HIST_W,), jnp.float32),
        pltpu.VMEM((_NBINS,), jnp.float32),
        pltpu.SemaphoreType.DMA,
        pltpu.SemaphoreType.DMA,
    ],
    compiler_params=pltpu.CompilerParams(needs_layout_passes=False),
)
def _hist_sc(x_hbm, out_hbm, br0, bg0, bb0, br1, bg1, bb1, hist, ostage,
             sem0, sem1):
    cid = lax.axis_index("c")
    sid = lax.axis_index("s")
    wid = sid * _NC + cid
    img0 = wid * _IMGS_PER_W
    fbase = img0 * (_PIX * _C)

    zeros = jnp.zeros((_L,), jnp.float32)
    ones = jnp.ones((_L,), jnp.float32)
    iota = lax.iota(jnp.int32, _L)
    lane_base = iota * _NBINS

    def _zbody(i, carry):
        hist[pl.ds(pl.multiple_of(i * _L, _L), _L)] = zeros
        return carry
    lax.fori_loop(0, _HIST_W // _L, _zbody, 0)

    sems = (sem0, sem1)
    bufsets = ((br0, bg0, bb0), (br1, bg1, bb1))

    def _plane_off(c, plane):
        img_l = jnp.where(c >= _CHUNKS_PER_IMG, 1, 0)
        ci = c - img_l * _CHUNKS_PER_IMG
        return pl.multiple_of(
            fbase + img_l * (_PIX * _C) + plane * _PIX + ci * _CHUNK_PX,
            _CHUNK_PX)

    def _start(c, b):
        for plane, buf in enumerate(bufsets[b]):
            pltpu.make_async_copy(
                x_hbm.at[pl.ds(_plane_off(c, plane), _CHUNK_PX)],
                buf, sems[b]).start()

    def _wait(b):
        for buf in bufsets[b]:
            pltpu.make_async_copy(
                x_hbm.at[pl.ds(0, _CHUNK_PX)], buf, sems[b]).wait()

    def _process(c, b):
        _wait(b)
        br, bg, bb = bufsets[b]
        hbase = (lane_base - jnp.int32(0x4B000000)) + jnp.where(
            c >= _CHUNKS_PER_IMG, _L * _NBINS, 0)

        def _rnd(v):
            u = plsc.bitcast(v, jnp.uint32)
            return plsc.bitcast(
                (u + jnp.uint32(0x8000)) & jnp.uint32(0xFFFF0000),
                jnp.float32)

        @plsc.parallel_loop(0, _CHUNK_PX, step=2 * _L, unroll=4)
        def _gbody(o):
            o0 = pl.multiple_of(o, 2 * _L)
            o1 = pl.multiple_of(o0 + _L, _L)
            r0, r1 = _rnd(br[pl.ds(o0, _L)]), _rnd(br[pl.ds(o1, _L)])
            g0, g1 = _rnd(bg[pl.ds(o0, _L)]), _rnd(bg[pl.ds(o1, _L)])
            b0, b1 = _rnd(bb[pl.ds(o0, _L)]), _rnd(bb[pl.ds(o1, _L)])
            acc0 = (r0 * _W0 + g0 * _W1) + b0 * _W2
            acc1 = (r1 * _W0 + g1 * _W1) + b1 * _W2
            i0 = plsc.bitcast(acc0 + _MAGIC, jnp.int32) + hbase
            i1 = plsc.bitcast(acc1 + _MAGIC, jnp.int32) + hbase
            plsc.addupdate_scatter(hist, [i0], ones)
            plsc.addupdate_scatter(hist, [i1], ones)

    _start(0, 0)

    def _cbody(c2, carry):
        c = c2 * 2
        _start(c + 1, 1)
        _process(c, 0)

        @pl.when(c2 < _TOTAL_CHUNKS // 2 - 1)
        def _():
            _start(c + 2, 0)
        _process(c + 1, 1)
        return carry
    lax.fori_loop(0, _TOTAL_CHUNKS // 2, _cbody, 0)

    for img_local in range(_IMGS_PER_W):
        def _rbody(g, carry, img_local=img_local):
            gb = pl.multiple_of(g * _L, _L)
            acc = zeros
            for l in range(_L):
                off = pl.multiple_of(
                    img_local * (_L * _NBINS) + l * _NBINS + gb, _L)
                acc = acc + hist[pl.ds(off, _L)]
            ostage[pl.ds(gb, _L)] = acc * _INV
            return carry
        lax.fori_loop(0, _NBINS // _L, _rbody, 0)
        oimg = img0 + img_local
        pltpu.sync_copy(
            ostage, out_hbm.at[pl.ds(pl.multiple_of(oimg * _NBINS, _NBINS),
                                     _NBINS)])


def kernel(inputs):
    x = inputs.astype(jnp.float32)
    y = x.transpose(0, 3, 1, 2)
    y = y.reshape(_B, _C, _H // 8, 8, _W // 128, 128)
    y = y.transpose(0, 1, 2, 4, 3, 5)
    flat = y.reshape(-1)
    out = _hist_sc(flat)
    return out.reshape(_B, _NBINS)


# speedup vs baseline: 175.1049x; 1.0146x over previous
import functools

import jax
import jax.numpy as jnp
import numpy as np
from jax import lax
from jax.experimental import pallas as pl
from jax.experimental.pallas import tpu as pltpu
from jax.experimental.pallas import tpu_sc as plsc

_NBINS = 256
_B, _H, _W, _C = 64, 512, 512, 3
_PIX = _H * _W
_NC, _NS, _L = 2, 16, 16
_NW = _NC * _NS
_IMGS_PER_W = _B // _NW
_CHUNK_PX = 16384
_CHUNKS_PER_IMG = _PIX // _CHUNK_PX
_TOTAL_CHUNKS = _CHUNKS_PER_IMG * _IMGS_PER_W
_PAIRS = _CHUNK_PX // (2 * _L)
_HIST_W = _IMGS_PER_W * _L * _NBINS
_INV = 1.0 / float(_PIX)


def _bf16_rne(v: float) -> float:
    u = np.float32(v).view(np.uint32)
    u = (u + np.uint32(0x7FFF) + ((u >> np.uint32(16)) & np.uint32(1)))
    return float((u & np.uint32(0xFFFF0000)).view(np.float32))


_W0 = _bf16_rne(0.2989) * _NBINS
_W1 = _bf16_rne(0.5870) * _NBINS
_W2 = _bf16_rne(0.1140) * _NBINS

_MAGIC = float(2.0 ** 23) - 0.5

_mesh = plsc.VectorSubcoreMesh(
    core_axis_name="c", subcore_axis_name="s",
    num_cores=_NC, num_subcores=_NS)


@functools.partial(
    pl.kernel,
    out_type=jax.ShapeDtypeStruct((_B * _NBINS,), jnp.float32),
    mesh=_mesh,
    scratch_types=[
        pltpu.VMEM((_CHUNK_PX,), jnp.float32),
        pltpu.VMEM((_CHUNK_PX,), jnp.float32),
        pltpu.VMEM((_CHUNK_PX,), jnp.float32),
        pltpu.VMEM((_CHUNK_PX,), jnp.float32),
        pltpu.VMEM((_CHUNK_PX,), jnp.float32),
        pltpu.VMEM((_CHUNK_PX,), jnp.float32),
        pltpu.VMEM((_HIST_W,), jnp.float32),
        pltpu.VMEM((_NBINS,), jnp.float32),
        pltpu.SemaphoreType.DMA,
        pltpu.SemaphoreType.DMA,
    ],
    compiler_params=pltpu.CompilerParams(needs_layout_passes=False),
)
def _hist_sc(x_hbm, out_hbm, br0, bg0, bb0, br1, bg1, bb1, hist, ostage,
             sem0, sem1):
    cid = lax.axis_index("c")
    sid = lax.axis_index("s")
    wid = sid * _NC + cid
    img0 = wid * _IMGS_PER_W
    fbase = img0 * (_PIX * _C)

    zeros = jnp.zeros((_L,), jnp.float32)
    ones = jnp.ones((_L,), jnp.float32)
    iota = lax.iota(jnp.int32, _L)
    lane_base = iota * _NBINS

    def _zbody(i, carry):
        hist[pl.ds(pl.multiple_of(i * _L, _L), _L)] = zeros
        return carry
    lax.fori_loop(0, _HIST_W // _L, _zbody, 0)

    sems = (sem0, sem1)
    bufsets = ((br0, bg0, bb0), (br1, bg1, bb1))

    def _plane_off(c, plane):
        img_l = jnp.where(c >= _CHUNKS_PER_IMG, 1, 0)
        ci = c - img_l * _CHUNKS_PER_IMG
        return pl.multiple_of(
            fbase + img_l * (_PIX * _C) + plane * _PIX + ci * _CHUNK_PX,
            _CHUNK_PX)

    def _start(c, b):
        for plane, buf in enumerate(bufsets[b]):
            pltpu.make_async_copy(
                x_hbm.at[pl.ds(_plane_off(c, plane), _CHUNK_PX)],
                buf, sems[b]).start()

    def _wait(b):
        for buf in bufsets[b]:
            pltpu.make_async_copy(
                x_hbm.at[pl.ds(0, _CHUNK_PX)], buf, sems[b]).wait()

    def _process(c, b):
        _wait(b)
        br, bg, bb = bufsets[b]
        hbase = (lane_base - jnp.int32(0x4B000000)) + jnp.where(
            c >= _CHUNKS_PER_IMG, _L * _NBINS, 0)

        def _rnd(v):
            u = plsc.bitcast(v, jnp.uint32)
            return plsc.bitcast(
                (u + jnp.uint32(0x8000)) & jnp.uint32(0xFFFF0000),
                jnp.float32)

        @plsc.parallel_loop(0, _CHUNK_PX, step=2 * _L, unroll=4)
        def _gbody(o):
            o0 = pl.multiple_of(o, 2 * _L)
            o1 = pl.multiple_of(o0 + _L, _L)
            r0, r1 = _rnd(br[pl.ds(o0, _L)]), _rnd(br[pl.ds(o1, _L)])
            g0, g1 = _rnd(bg[pl.ds(o0, _L)]), _rnd(bg[pl.ds(o1, _L)])
            b0, b1 = _rnd(bb[pl.ds(o0, _L)]), _rnd(bb[pl.ds(o1, _L)])
            acc0 = (r0 * _W0 + g0 * _W1) + b0 * _W2
            acc1 = (r1 * _W0 + g1 * _W1) + b1 * _W2
            i0 = plsc.bitcast(acc0 + _MAGIC, jnp.int32) + hbase
            i1 = plsc.bitcast(acc1 + _MAGIC, jnp.int32) + hbase
            plsc.addupdate_scatter(hist, [i0], ones)
            plsc.addupdate_scatter(hist, [i1], ones)

    _start(0, 0)

    def _cbody(c2, carry):
        c = c2 * 2
        _start(c + 1, 1)
        _process(c, 0)

        @pl.when(c2 < _TOTAL_CHUNKS // 2 - 1)
        def _():
            _start(c + 2, 0)
        _process(c + 1, 1)
        return carry
    lax.fori_loop(0, _TOTAL_CHUNKS // 2, _cbody, 0)

    for img_local in range(_IMGS_PER_W):
        def _rbody(g, carry, img_local=img_local):
            gb = pl.multiple_of(g * _L, _L)
            acc = zeros
            for l in range(_L):
                off = pl.multiple_of(
                    img_local * (_L * _NBINS) + l * _NBINS + gb, _L)
                acc = acc + hist[pl.ds(off, _L)]
            ostage[pl.ds(gb, _L)] = acc * _INV
            return carry
        lax.fori_loop(0, _NBINS // _L, _rbody, 0)
        oimg = img0 + img_local
        base0 = pl.multiple_of(
            (oimg >> 3) * 2048 + (oimg & 7) * 128, 128)
        pltpu.sync_copy(ostage.at[pl.ds(0, 128)],
                        out_hbm.at[pl.ds(base0, 128)])
        pltpu.sync_copy(ostage.at[pl.ds(128, 128)],
                        out_hbm.at[pl.ds(base0 + 1024, 128)])


def kernel(inputs):
    x = inputs.astype(jnp.float32)
    y = x.transpose(0, 3, 1, 2)
    y = y.reshape(_B, _C, _H // 8, 8, _W // 128, 128)
    y = y.transpose(0, 1, 2, 4, 3, 5)
    flat = y.reshape(-1)
    out = _hist_sc(flat)
    o = out.reshape(_B // 8, _NBINS // 128, 8, 128)
    o = o.transpose(0, 2, 1, 3)
    return o.reshape(_B, _NBINS)
